# baseline (speedup 1.0000x reference)
import functools

import jax
import jax.numpy as jnp
from jax import lax
from jax.experimental import pallas as pl
from jax.experimental.pallas import tpu as pltpu
from jax.experimental.pallas import tpu_sc as plsc

PREFIX_LENGTH = 128
NUM_LAYERS = 24
HIDDEN_SIZE = 1024
BATCH = 16
EMBED_DIM = NUM_LAYERS * HIDDEN_SIZE
B = BATCH * PREFIX_LENGTH
V = PREFIX_LENGTH

NC, NS = 2, 16
NPHASE = 2
Q = EMBED_DIM // (NC * NPHASE)
RPT = B // NS
VPT = V // NS
NVEC = RPT // 16

LP = [l for l in range(16) if l % 4 != 3]
LD = [l for l in range(16) if l % 4 == 3]
NP = len(LP)
NBUF = 6
DIST = 4
ND = len(LD)

_mesh = plsc.VectorSubcoreMesh(core_axis_name="c", subcore_axis_name="s")


@functools.partial(
    pl.kernel,
    mesh=_mesh,
    out_type=jax.ShapeDtypeStruct((B, EMBED_DIM), jnp.float32),
    scratch_types=(
        [pltpu.VMEM((RPT,), jnp.int32)]
        + [pltpu.VMEM((1, Q), jnp.float32) for _ in range(NBUF)]
        + [pltpu.VMEM_SHARED((V, Q), jnp.float32)]
        + [pltpu.SemaphoreType.DMA for _ in range(2 * NBUF + ND)]
    ),
)
def _gather_kernel(idx_hbm, table_hbm, out_hbm, idx_v, *rest):
    bufs = rest[:NBUF]
    shared_tab = rest[NBUF]
    gsem = rest[NBUF + 1:NBUF + 1 + NBUF]
    ssem = rest[NBUF + 1 + NBUF:NBUF + 1 + 2 * NBUF]
    dsem = rest[NBUF + 1 + 2 * NBUF:]
    c = lax.axis_index("c")
    s = lax.axis_index("s")

    pltpu.sync_copy(idx_hbm.at[s], idx_v)
    row_base = s * RPT

    def gather(v, b):
        pltpu.async_copy(shared_tab.at[pl.ds(v, 1)], bufs[b], gsem[b])

    for p in range(NPHASE):
        col0 = c * (NPHASE * Q) + p * Q
        out_at = lambda k: out_hbm.at[pl.ds(row_base + k, 1), pl.ds(col0, Q)]

        if p > 0:
            plsc.subcore_barrier()
        pltpu.sync_copy(
            table_hbm.at[pl.ds(VPT * s, VPT), pl.ds(col0, Q)],
            shared_tab.at[pl.ds(VPT * s, VPT)],
        )
        plsc.subcore_barrier()

        vec0 = idx_v[pl.ds(0, 16)]
        for o in range(DIST):
            gather(vec0[LP[o]], o % NBUF)

        def body(j, carry):
            vecs = idx_v[pl.ds(j * 16, 16)]
            vecs2 = idx_v[pl.ds(lax.min(j * 16 + 16, RPT - 16), 16)]

            def _wait_store(b2, krel):
                pltpu.make_async_copy(
                    bufs[b2], out_at(j * 16 + krel), ssem[b2]
                ).wait()

            for od, ld in enumerate(LD):
                def _wait_prev_direct(od=od, ld=ld):
                    pltpu.make_async_copy(
                        shared_tab.at[pl.ds(0, 1)],
                        out_at((j - 1) * 16 + ld), dsem[od]
                    ).wait()
                pl.when(j > 0)(_wait_prev_direct)
                pltpu.async_copy(
                    shared_tab.at[pl.ds(vecs[ld], 1)],
                    out_at(j * 16 + ld), dsem[od])

            for o in range(NP):
                b = o % NBUF
                k = j * 16 + LP[o]

                pltpu.make_async_copy(
                    shared_tab.at[pl.ds(0, 1)], bufs[b], gsem[b]
                ).wait()
                pltpu.async_copy(bufs[b], out_at(k), ssem[b])

                b2 = (o + DIST) % NBUF
                if o + DIST < NP:
                    if o + DIST >= NBUF:
                        _wait_store(b2, LP[o + DIST - NBUF])
                    else:
                        pl.when(j > 0)(
                            functools.partial(
                                _wait_store, b2, LP[o + DIST - NBUF] - 16))
                    gather(vecs[LP[o + DIST]], b2)
                else:
                    @pl.when(j < NVEC - 1)
                    def _prefetch_next_group(o=o, b2=b2):
                        _wait_store(b2, LP[o + DIST - NBUF])
                        gather(vecs2[LP[o + DIST - NP]], b2)
            return carry

        lax.fori_loop(0, NVEC, body, 0)
        for o in range(NP - NBUF, NP):
            pltpu.make_async_copy(
                bufs[o % NBUF], out_at((NVEC - 1) * 16 + LP[o]),
                ssem[o % NBUF]
            ).wait()
        for od, ld in enumerate(LD):
            pltpu.make_async_copy(
                shared_tab.at[pl.ds(0, 1)],
                out_at((NVEC - 1) * 16 + ld), dsem[od]
            ).wait()


def kernel(prefix, table):
    idx = prefix.astype(jnp.int32).reshape(NS, RPT)
    out = _gather_kernel(idx, table)
    return out.reshape(BATCH, PREFIX_LENGTH, EMBED_DIM)
